# baseline (speedup 1.0000x reference)
import functools

import jax
import jax.numpy as jnp
from jax import lax
from jax.experimental import pallas as pl
from jax.experimental.pallas import tpu as pltpu
from jax.experimental.pallas import tpu_sc as plsc

N = 10000
R = 8
D = 128
E = 320000
B = 1024

NC = 2
NS = 16
NW = NC * NS
CHUNK = 16
NPASS = D // CHUNK
SROWS = R * N
ACC_ROWS = SROWS
ENC_ROWS = SROWS + 8
TILE_ROWS = ACC_ROWS // NS
ZROWS = TILE_ROWS // 8

EBATCH = 128
EP = 327680
EROWS = EP // EBATCH
TBATCH = EROWS // NW
CCHUNK = 5
NCHUNK = TBATCH // CCHUNK
ZROWS_BUF = 125
EPS = 1e-12


def _norm_body(x_ref, o_ref):
    x = x_ref[...]
    n = jnp.sqrt(jnp.sum(x * x, axis=1, keepdims=True))
    o_ref[...] = x / jnp.maximum(n, EPS)


def _normalize(emb):
    return pl.pallas_call(
        _norm_body,
        grid=(10,),
        in_specs=[pl.BlockSpec((N // 10, D), lambda i: (i, 0))],
        out_specs=pl.BlockSpec((N // 10, D), lambda i: (i, 0)),
        out_shape=jax.ShapeDtypeStruct((N, D), jnp.float32),
    )(emb)


def _sc_body(enc_hbm, src_hbm, rel_hbm, dst_hbm, hidx_hbm, en_hbm,
             acc_out, hrows_out,
             acc, sidx_all, dbuf, gidx, rows, zbuf, hidx, hrow,
             gsem, ssem, lsem):
    cid = lax.axis_index("c")
    sid = lax.axis_index("s")
    wid = cid * NS + sid
    myrow = wid * TBATCH

    hper = B // NW
    pltpu.sync_copy(hidx_hbm.at[pl.ds(wid * hper, hper)], hidx)
    pltpu.async_copy(en_hbm.at[hidx], hrow, gsem).wait()
    pltpu.sync_copy(hrow, hrows_out.at[pl.ds(wid * hper, hper)])

    def _mk_sidx(ch, _):
        sl5 = pl.ds(ch * CCHUNK, CCHUNK)
        a = pltpu.async_copy(src_hbm.at[pl.ds(myrow + ch * CCHUNK, CCHUNK)],
                             gidx.at[0], lsem)
        b = pltpu.async_copy(rel_hbm.at[pl.ds(myrow + ch * CCHUNK, CCHUNK)],
                             gidx.at[1], lsem)
        a.wait()
        b.wait()
        for j in range(CCHUNK):
            for l in range(EBATCH // 16):
                sl = pl.ds(l * 16, 16)
                sidx_all[ch * CCHUNK + j, sl] = gidx[1, j, sl] * N + gidx[0, j, sl]
        return 0
    lax.fori_loop(0, NCHUNK, _mk_sidx, 0)

    def _z(i, _):
        zbuf[i] = jnp.zeros((16,), jnp.float32)
        return 0
    lax.fori_loop(0, ZROWS_BUF, _z, 0)

    def _load_dst(ch, par):
        return pltpu.async_copy(
            dst_hbm.at[pl.ds(myrow + ch * CCHUNK, CCHUNK)], dbuf.at[par], lsem)

    def _pass(c, _):
        zcps = [pltpu.async_copy(
            zbuf, acc.at[pl.ds(sid * TILE_ROWS + q * ZROWS_BUF, ZROWS_BUF)],
            gsem) for q in range(TILE_ROWS // ZROWS_BUF)]
        for cp in zcps:
            cp.wait()
        plsc.subcore_barrier()

        def _gidx_of(ch, par):
            for j in range(CCHUNK):
                for l in range(EBATCH // 16):
                    sl = pl.ds(l * 16, 16)
                    gidx[par, j, sl] = jnp.minimum(
                        dbuf[par, j, sl] + c * N, ENC_ROWS - 1)

        def _fire_gathers(par):
            return [pltpu.async_copy(enc_hbm.at[gidx.at[par, j]],
                                     rows.at[par, j], gsem)
                    for j in range(CCHUNK)]

        def _fire_scatters(ch, par):
            return [pltpu.async_copy(
                rows.at[par, j], acc.at[sidx_all.at[ch * CCHUNK + j]],
                ssem, add=True) for j in range(CCHUNK)]

        sc_pend = [None, None]
        ld = _load_dst(0, 0)
        for ch in range(NCHUNK):
            par = ch % 2
            ld.wait()
            if ch + 1 < NCHUNK:
                ld = _load_dst(ch + 1, 1 - par)
            _gidx_of(ch, par)
            if sc_pend[par] is not None:
                for cp in sc_pend[par]:
                    cp.wait()
            g = _fire_gathers(par)
            for cp in g:
                cp.wait()
            sc_pend[par] = _fire_scatters(ch, par)
        for p in range(2):
            if sc_pend[p] is not None:
                for cp in sc_pend[p]:
                    cp.wait()

        plsc.subcore_barrier()
        pltpu.sync_copy(acc.at[pl.ds(sid * TILE_ROWS, TILE_ROWS)],
                        acc_out.at[cid, c, pl.ds(sid * TILE_ROWS, TILE_ROWS)])
        return 0

    lax.fori_loop(0, NPASS, _pass, 0)


_sc_phase1 = functools.partial(
    pl.kernel,
    out_type=[
        jax.ShapeDtypeStruct((NC, NPASS, ACC_ROWS, CHUNK), jnp.float32),
        jax.ShapeDtypeStruct((B, D), jnp.float32),
    ],
    mesh=plsc.VectorSubcoreMesh(core_axis_name="c", subcore_axis_name="s"),
    scratch_types=[
        pltpu.VMEM_SHARED((ACC_ROWS, CHUNK), jnp.float32),
        pltpu.VMEM((TBATCH, EBATCH), jnp.int32),
        pltpu.VMEM((2, CCHUNK, EBATCH), jnp.int32),
        pltpu.VMEM((2, CCHUNK, EBATCH), jnp.int32),
        pltpu.VMEM((2, CCHUNK, EBATCH, CHUNK), jnp.float32),
        pltpu.VMEM((ZROWS_BUF, CHUNK), jnp.float32),
        pltpu.VMEM((B // NW,), jnp.int32),
        pltpu.VMEM((B // NW, D), jnp.float32),
        pltpu.SemaphoreType.DMA,
        pltpu.SemaphoreType.DMA,
        pltpu.SemaphoreType.DMA,
    ],
    compiler_params=pltpu.CompilerParams(use_tc_tiling_on_sc=False),
)(_sc_body)


def _red_body(acc_ref, en_ref, out_ref):
    s = pl.program_id(2)
    z = acc_ref[0, 0]
    e = en_ref[...]
    prod = lax.dot_general(z, e, (((0,), (0,)), ((), ())),
                           preferred_element_type=jnp.float32)

    @pl.when(s == 0)
    def _():
        out_ref[...] = prod[None, None]

    @pl.when(s != 0)
    def _():
        out_ref[...] += prod[None, None]


def _reduce(acc_all, en):
    return pl.pallas_call(
        _red_body,
        grid=(R, NPASS, NC),
        in_specs=[
            pl.BlockSpec((1, 1, N, CHUNK), lambda r, c, s: (s, c, r, 0)),
            pl.BlockSpec((N, D), lambda r, c, s: (0, 0)),
        ],
        out_specs=pl.BlockSpec((1, 1, CHUNK, D), lambda r, c, s: (r, c, 0, 0)),
        out_shape=jax.ShapeDtypeStruct((R, NPASS, CHUNK, D), jnp.float32),
    )(acc_all, en)


def _pred_body(h_ref, ridx_ref, rall_ref, out_ref):
    h = h_ref[...]
    ridx = ridx_ref[...]
    acc = jnp.zeros((B, D), jnp.float32)
    for r in range(R):
        pr = jnp.dot(h, rall_ref[r], preferred_element_type=jnp.float32)
        acc = acc + jnp.where(ridx == r, pr, 0.0)
    n = jnp.sqrt(jnp.sum(acc * acc, axis=1, keepdims=True))
    out_ref[...] = acc / jnp.maximum(n, EPS)


def _predict(hrows, ridx2, rall):
    return pl.pallas_call(
        _pred_body,
        in_specs=[
            pl.BlockSpec(memory_space=pltpu.VMEM),
            pl.BlockSpec(memory_space=pltpu.VMEM),
            pl.BlockSpec(memory_space=pltpu.VMEM),
        ],
        out_specs=pl.BlockSpec(memory_space=pltpu.VMEM),
        out_shape=jax.ShapeDtypeStruct((B, D), jnp.float32),
    )(hrows, ridx2, rall)


def kernel(h_idx, r_idx, edge_index, edge_rel, entity_emb):
    src = edge_index[0].astype(jnp.int32)
    dst = edge_index[1].astype(jnp.int32)
    rel = edge_rel.astype(jnp.int32)

    en = _normalize(entity_emb)
    enc = jnp.concatenate(
        [en.reshape(N, NPASS, CHUNK).transpose(1, 0, 2).reshape(NPASS * N, CHUNK),
         jnp.zeros((ENC_ROWS - NPASS * N, CHUNK), jnp.float32)])

    npad = EP - E
    srcp = jnp.concatenate([src, jnp.zeros((npad,), jnp.int32)]).reshape(EROWS, EBATCH)
    relp = jnp.concatenate([rel, jnp.zeros((npad,), jnp.int32)]).reshape(EROWS, EBATCH)
    dstp = jnp.concatenate([dst, jnp.full((npad,), SROWS, jnp.int32)]).reshape(EROWS, EBATCH)

    acc_all, hrows = _sc_phase1(enc, srcp, relp, dstp, h_idx.astype(jnp.int32), en)

    rblk = _reduce(acc_all, en)
    rall = rblk.transpose(0, 3, 1, 2).reshape(R, D, D)

    return _predict(hrows, r_idx.astype(jnp.int32).reshape(B, 1), rall)


# speedup vs baseline: 1.0064x; 1.0064x over previous
import functools

import jax
import jax.numpy as jnp
from jax import lax
from jax.experimental import pallas as pl
from jax.experimental.pallas import tpu as pltpu
from jax.experimental.pallas import tpu_sc as plsc

N = 10000
R = 8
D = 128
E = 320000
B = 1024

NC = 2
NS = 16
NW = NC * NS
CHUNK = 16
NPASS = D // CHUNK
SROWS = R * N
ACC_ROWS = SROWS
ENC_ROWS = SROWS + 8
TILE_ROWS = ACC_ROWS // NS
ZROWS = TILE_ROWS // 8

DO_GATHER = True
DO_SCATTER = False
DO_ZERO = True
DO_WRITEOUT = True

EBATCH = 128
EP = 327680
EROWS = EP // EBATCH
TBATCH = EROWS // NW
CCHUNK = 5
NCHUNK = TBATCH // CCHUNK
ZROWS_BUF = 125
EPS = 1e-12


def _norm_body(x_ref, o_ref):
    x = x_ref[...]
    n = jnp.sqrt(jnp.sum(x * x, axis=1, keepdims=True))
    o_ref[...] = x / jnp.maximum(n, EPS)


def _normalize(emb):
    return pl.pallas_call(
        _norm_body,
        grid=(10,),
        in_specs=[pl.BlockSpec((N // 10, D), lambda i: (i, 0))],
        out_specs=pl.BlockSpec((N // 10, D), lambda i: (i, 0)),
        out_shape=jax.ShapeDtypeStruct((N, D), jnp.float32),
    )(emb)


def _sc_body(enc_hbm, src_hbm, rel_hbm, dst_hbm, hidx_hbm, en_hbm,
             acc_out, hrows_out,
             acc, sidx_all, dbuf, gidx, rows, zbuf, hidx, hrow,
             gsem, ssem, lsem):
    cid = lax.axis_index("c")
    sid = lax.axis_index("s")
    wid = cid * NS + sid
    myrow = wid * TBATCH

    hper = B // NW
    pltpu.sync_copy(hidx_hbm.at[pl.ds(wid * hper, hper)], hidx)
    pltpu.async_copy(en_hbm.at[hidx], hrow, gsem).wait()
    pltpu.sync_copy(hrow, hrows_out.at[pl.ds(wid * hper, hper)])

    def _mk_sidx(ch, _):
        sl5 = pl.ds(ch * CCHUNK, CCHUNK)
        a = pltpu.async_copy(src_hbm.at[pl.ds(myrow + ch * CCHUNK, CCHUNK)],
                             gidx.at[0], lsem)
        b = pltpu.async_copy(rel_hbm.at[pl.ds(myrow + ch * CCHUNK, CCHUNK)],
                             gidx.at[1], lsem)
        a.wait()
        b.wait()
        for j in range(CCHUNK):
            for l in range(EBATCH // 16):
                sl = pl.ds(l * 16, 16)
                sidx_all[ch * CCHUNK + j, sl] = gidx[1, j, sl] * N + gidx[0, j, sl]
        return 0
    lax.fori_loop(0, NCHUNK, _mk_sidx, 0)

    def _z(i, _):
        zbuf[i] = jnp.zeros((16,), jnp.float32)
        return 0
    lax.fori_loop(0, ZROWS_BUF, _z, 0)

    def _load_dst(ch, par):
        return pltpu.async_copy(
            dst_hbm.at[pl.ds(myrow + ch * CCHUNK, CCHUNK)], dbuf.at[par], lsem)

    def _pass(c, _):
        if DO_ZERO:
            zcps = [pltpu.async_copy(
                zbuf, acc.at[pl.ds(sid * TILE_ROWS + q * ZROWS_BUF, ZROWS_BUF)],
                gsem) for q in range(TILE_ROWS // ZROWS_BUF)]
            for cp in zcps:
                cp.wait()
        plsc.subcore_barrier()

        def _gidx_of(ch, par):
            for j in range(CCHUNK):
                for l in range(EBATCH // 16):
                    sl = pl.ds(l * 16, 16)
                    gidx[par, j, sl] = jnp.minimum(
                        dbuf[par, j, sl] + c * N, ENC_ROWS - 1)

        def _fire_gathers(par):
            return [pltpu.async_copy(enc_hbm.at[gidx.at[par, j]],
                                     rows.at[par, j], gsem)
                    for j in range(CCHUNK)]

        def _fire_scatters(ch, par):
            return [pltpu.async_copy(
                rows.at[par, j], acc.at[sidx_all.at[ch * CCHUNK + j]],
                ssem, add=True) for j in range(CCHUNK)]

        sc_pend = [None, None]
        ld = _load_dst(0, 0)
        for ch in range(NCHUNK):
            par = ch % 2
            ld.wait()
            if ch + 1 < NCHUNK:
                ld = _load_dst(ch + 1, 1 - par)
            _gidx_of(ch, par)
            if sc_pend[par] is not None:
                for cp in sc_pend[par]:
                    cp.wait()
            if DO_GATHER:
                g = _fire_gathers(par)
                for cp in g:
                    cp.wait()
            if DO_SCATTER:
                sc_pend[par] = _fire_scatters(ch, par)
        for p in range(2):
            if sc_pend[p] is not None:
                for cp in sc_pend[p]:
                    cp.wait()

        plsc.subcore_barrier()
        if DO_WRITEOUT:
            pltpu.sync_copy(acc.at[pl.ds(sid * TILE_ROWS, TILE_ROWS)],
                            acc_out.at[cid, c, pl.ds(sid * TILE_ROWS, TILE_ROWS)])
        return 0

    lax.fori_loop(0, NPASS, _pass, 0)


_sc_phase1 = functools.partial(
    pl.kernel,
    out_type=[
        jax.ShapeDtypeStruct((NC, NPASS, ACC_ROWS, CHUNK), jnp.float32),
        jax.ShapeDtypeStruct((B, D), jnp.float32),
    ],
    mesh=plsc.VectorSubcoreMesh(core_axis_name="c", subcore_axis_name="s"),
    scratch_types=[
        pltpu.VMEM_SHARED((ACC_ROWS, CHUNK), jnp.float32),
        pltpu.VMEM((TBATCH, EBATCH), jnp.int32),
        pltpu.VMEM((2, CCHUNK, EBATCH), jnp.int32),
        pltpu.VMEM((2, CCHUNK, EBATCH), jnp.int32),
        pltpu.VMEM((2, CCHUNK, EBATCH, CHUNK), jnp.float32),
        pltpu.VMEM((ZROWS_BUF, CHUNK), jnp.float32),
        pltpu.VMEM((B // NW,), jnp.int32),
        pltpu.VMEM((B // NW, D), jnp.float32),
        pltpu.SemaphoreType.DMA,
        pltpu.SemaphoreType.DMA,
        pltpu.SemaphoreType.DMA,
    ],
    compiler_params=pltpu.CompilerParams(use_tc_tiling_on_sc=False),
)(_sc_body)


def _red_body(acc_ref, en_ref, out_ref):
    s = pl.program_id(2)
    z = acc_ref[0, 0]
    e = en_ref[...]
    prod = lax.dot_general(z, e, (((0,), (0,)), ((), ())),
                           preferred_element_type=jnp.float32)

    @pl.when(s == 0)
    def _():
        out_ref[...] = prod[None, None]

    @pl.when(s != 0)
    def _():
        out_ref[...] += prod[None, None]


def _reduce(acc_all, en):
    return pl.pallas_call(
        _red_body,
        grid=(R, NPASS, NC),
        in_specs=[
            pl.BlockSpec((1, 1, N, CHUNK), lambda r, c, s: (s, c, r, 0)),
            pl.BlockSpec((N, D), lambda r, c, s: (0, 0)),
        ],
        out_specs=pl.BlockSpec((1, 1, CHUNK, D), lambda r, c, s: (r, c, 0, 0)),
        out_shape=jax.ShapeDtypeStruct((R, NPASS, CHUNK, D), jnp.float32),
    )(acc_all, en)


def _pred_body(h_ref, ridx_ref, rall_ref, out_ref):
    h = h_ref[...]
    ridx = ridx_ref[...]
    acc = jnp.zeros((B, D), jnp.float32)
    for r in range(R):
        pr = jnp.dot(h, rall_ref[r], preferred_element_type=jnp.float32)
        acc = acc + jnp.where(ridx == r, pr, 0.0)
    n = jnp.sqrt(jnp.sum(acc * acc, axis=1, keepdims=True))
    out_ref[...] = acc / jnp.maximum(n, EPS)


def _predict(hrows, ridx2, rall):
    return pl.pallas_call(
        _pred_body,
        in_specs=[
            pl.BlockSpec(memory_space=pltpu.VMEM),
            pl.BlockSpec(memory_space=pltpu.VMEM),
            pl.BlockSpec(memory_space=pltpu.VMEM),
        ],
        out_specs=pl.BlockSpec(memory_space=pltpu.VMEM),
        out_shape=jax.ShapeDtypeStruct((B, D), jnp.float32),
    )(hrows, ridx2, rall)


def kernel(h_idx, r_idx, edge_index, edge_rel, entity_emb):
    src = edge_index[0].astype(jnp.int32)
    dst = edge_index[1].astype(jnp.int32)
    rel = edge_rel.astype(jnp.int32)

    en = _normalize(entity_emb)
    enc = jnp.concatenate(
        [en.reshape(N, NPASS, CHUNK).transpose(1, 0, 2).reshape(NPASS * N, CHUNK),
         jnp.zeros((ENC_ROWS - NPASS * N, CHUNK), jnp.float32)])

    npad = EP - E
    srcp = jnp.concatenate([src, jnp.zeros((npad,), jnp.int32)]).reshape(EROWS, EBATCH)
    relp = jnp.concatenate([rel, jnp.zeros((npad,), jnp.int32)]).reshape(EROWS, EBATCH)
    dstp = jnp.concatenate([dst, jnp.full((npad,), SROWS, jnp.int32)]).reshape(EROWS, EBATCH)

    acc_all, hrows = _sc_phase1(enc, srcp, relp, dstp, h_idx.astype(jnp.int32), en)

    rblk = _reduce(acc_all, en)
    rall = rblk.transpose(0, 3, 1, 2).reshape(R, D, D)

    return _predict(hrows, r_idx.astype(jnp.int32).reshape(B, 1), rall)


# speedup vs baseline: 1.9376x; 1.9253x over previous
import functools

import jax
import jax.numpy as jnp
from jax import lax
from jax.experimental import pallas as pl
from jax.experimental.pallas import tpu as pltpu
from jax.experimental.pallas import tpu_sc as plsc

N = 10000
R = 8
D = 128
E = 320000
B = 1024

NC = 2
NS = 16
NW = NC * NS
CHUNK = 32
NPASS = D // CHUNK
SROWS = R * N
ACC_ROWS = SROWS
ENC_ROWS = NPASS * N + 8
TILE_ROWS = ACC_ROWS // NS
ZROWS = TILE_ROWS // 8

DO_GATHER = True
DO_SCATTER = True
DO_ZERO = True
DO_WRITEOUT = True

EBATCH = 128
EP = 327680
EROWS = EP // EBATCH
TBATCH = EROWS // NW
CCHUNK = 5
NCHUNK = TBATCH // CCHUNK
ZROWS_BUF = 125
EPS = 1e-12
ADT = jnp.bfloat16


def _norm_body(x_ref, o_ref):
    x = x_ref[...]
    n = jnp.sqrt(jnp.sum(x * x, axis=1, keepdims=True))
    o_ref[...] = x / jnp.maximum(n, EPS)


def _normalize(emb):
    return pl.pallas_call(
        _norm_body,
        grid=(10,),
        in_specs=[pl.BlockSpec((N // 10, D), lambda i: (i, 0))],
        out_specs=pl.BlockSpec((N // 10, D), lambda i: (i, 0)),
        out_shape=jax.ShapeDtypeStruct((N, D), jnp.float32),
    )(emb)


def _sc_body(enc_hbm, src_hbm, rel_hbm, dst_hbm, hidx_hbm, en_hbm,
             acc_out, hrows_out,
             acc, sidx_all, dbuf, gidx, rows, zbuf, hidx, hrow,
             gsem, ssem, lsem):
    cid = lax.axis_index("c")
    sid = lax.axis_index("s")
    wid = cid * NS + sid
    myrow = wid * TBATCH

    hper = B // NW
    pltpu.sync_copy(hidx_hbm.at[pl.ds(wid * hper, hper)], hidx)
    pltpu.async_copy(en_hbm.at[hidx], hrow, gsem).wait()
    pltpu.sync_copy(hrow, hrows_out.at[pl.ds(wid * hper, hper)])

    def _mk_sidx(ch, _):
        sl5 = pl.ds(ch * CCHUNK, CCHUNK)
        a = pltpu.async_copy(src_hbm.at[pl.ds(myrow + ch * CCHUNK, CCHUNK)],
                             gidx.at[0], lsem)
        b = pltpu.async_copy(rel_hbm.at[pl.ds(myrow + ch * CCHUNK, CCHUNK)],
                             gidx.at[1], lsem)
        a.wait()
        b.wait()
        for j in range(CCHUNK):
            for l in range(EBATCH // 16):
                sl = pl.ds(l * 16, 16)
                sidx_all[ch * CCHUNK + j, sl] = gidx[1, j, sl] * N + gidx[0, j, sl]
        return 0
    lax.fori_loop(0, NCHUNK, _mk_sidx, 0)

    def _z(i, _):
        zbuf[i] = jnp.zeros((CHUNK,), ADT)
        return 0
    lax.fori_loop(0, ZROWS_BUF, _z, 0)

    def _load_dst(ch, par):
        return pltpu.async_copy(
            dst_hbm.at[pl.ds(myrow + ch * CCHUNK, CCHUNK)], dbuf.at[par], lsem)

    def _pass(c, _):
        if DO_ZERO:
            zcps = [pltpu.async_copy(
                zbuf, acc.at[pl.ds(sid * TILE_ROWS + q * ZROWS_BUF, ZROWS_BUF)],
                gsem) for q in range(TILE_ROWS // ZROWS_BUF)]
            for cp in zcps:
                cp.wait()
        plsc.subcore_barrier()

        def _gidx_of(ch, par):
            for j in range(CCHUNK):
                for l in range(EBATCH // 16):
                    sl = pl.ds(l * 16, 16)
                    gidx[par, j, sl] = jnp.minimum(
                        dbuf[par, j, sl] + c * N, ENC_ROWS - 1)

        def _fire_gathers(par):
            return [pltpu.async_copy(enc_hbm.at[gidx.at[par, j]],
                                     rows.at[par, j], gsem)
                    for j in range(CCHUNK)]

        def _fire_scatters(ch, par):
            return [pltpu.async_copy(
                rows.at[par, j], acc.at[sidx_all.at[ch * CCHUNK + j]],
                ssem, add=True) for j in range(CCHUNK)]

        sc_pend = [None, None]
        ld = _load_dst(0, 0)
        for ch in range(NCHUNK):
            par = ch % 2
            ld.wait()
            if ch + 1 < NCHUNK:
                ld = _load_dst(ch + 1, 1 - par)
            _gidx_of(ch, par)
            if sc_pend[par] is not None:
                for cp in sc_pend[par]:
                    cp.wait()
            if DO_GATHER:
                g = _fire_gathers(par)
                for cp in g:
                    cp.wait()
            if DO_SCATTER:
                sc_pend[par] = _fire_scatters(ch, par)
        for p in range(2):
            if sc_pend[p] is not None:
                for cp in sc_pend[p]:
                    cp.wait()

        plsc.subcore_barrier()
        if DO_WRITEOUT:
            pltpu.sync_copy(acc.at[pl.ds(sid * TILE_ROWS, TILE_ROWS)],
                            acc_out.at[cid, c, pl.ds(sid * TILE_ROWS, TILE_ROWS)])
        return 0

    lax.fori_loop(0, NPASS, _pass, 0)


_sc_phase1 = functools.partial(
    pl.kernel,
    out_type=[
        jax.ShapeDtypeStruct((NC, NPASS, ACC_ROWS, CHUNK), ADT),
        jax.ShapeDtypeStruct((B, D), jnp.float32),
    ],
    mesh=plsc.VectorSubcoreMesh(core_axis_name="c", subcore_axis_name="s"),
    scratch_types=[
        pltpu.VMEM_SHARED((ACC_ROWS, CHUNK), ADT),
        pltpu.VMEM((TBATCH, EBATCH), jnp.int32),
        pltpu.VMEM((2, CCHUNK, EBATCH), jnp.int32),
        pltpu.VMEM((2, CCHUNK, EBATCH), jnp.int32),
        pltpu.VMEM((2, CCHUNK, EBATCH, CHUNK), ADT),
        pltpu.VMEM((ZROWS_BUF, CHUNK), ADT),
        pltpu.VMEM((B // NW,), jnp.int32),
        pltpu.VMEM((B // NW, D), jnp.float32),
        pltpu.SemaphoreType.DMA,
        pltpu.SemaphoreType.DMA,
        pltpu.SemaphoreType.DMA,
    ],
    compiler_params=pltpu.CompilerParams(use_tc_tiling_on_sc=False),
)(_sc_body)


def _red_body(acc_ref, en_ref, out_ref):
    s = pl.program_id(2)
    z = acc_ref[0, 0].astype(jnp.float32)
    e = en_ref[...]
    prod = lax.dot_general(z, e, (((0,), (0,)), ((), ())),
                           preferred_element_type=jnp.float32)

    @pl.when(s == 0)
    def _():
        out_ref[...] = prod[None, None]

    @pl.when(s != 0)
    def _():
        out_ref[...] += prod[None, None]


def _reduce(acc_all, en):
    return pl.pallas_call(
        _red_body,
        grid=(R, NPASS, NC),
        in_specs=[
            pl.BlockSpec((1, 1, N, CHUNK), lambda r, c, s: (s, c, r, 0)),
            pl.BlockSpec((N, D), lambda r, c, s: (0, 0)),
        ],
        out_specs=pl.BlockSpec((1, 1, CHUNK, D), lambda r, c, s: (r, c, 0, 0)),
        out_shape=jax.ShapeDtypeStruct((R, NPASS, CHUNK, D), jnp.float32),
    )(acc_all, en)


def _pred_body(h_ref, ridx_ref, rall_ref, out_ref):
    h = h_ref[...]
    ridx = ridx_ref[...]
    acc = jnp.zeros((B, D), jnp.float32)
    for r in range(R):
        pr = jnp.dot(h, rall_ref[r], preferred_element_type=jnp.float32)
        acc = acc + jnp.where(ridx == r, pr, 0.0)
    n = jnp.sqrt(jnp.sum(acc * acc, axis=1, keepdims=True))
    out_ref[...] = acc / jnp.maximum(n, EPS)


def _predict(hrows, ridx2, rall):
    return pl.pallas_call(
        _pred_body,
        in_specs=[
            pl.BlockSpec(memory_space=pltpu.VMEM),
            pl.BlockSpec(memory_space=pltpu.VMEM),
            pl.BlockSpec(memory_space=pltpu.VMEM),
        ],
        out_specs=pl.BlockSpec(memory_space=pltpu.VMEM),
        out_shape=jax.ShapeDtypeStruct((B, D), jnp.float32),
    )(hrows, ridx2, rall)


def kernel(h_idx, r_idx, edge_index, edge_rel, entity_emb):
    src = edge_index[0].astype(jnp.int32)
    dst = edge_index[1].astype(jnp.int32)
    rel = edge_rel.astype(jnp.int32)

    en = _normalize(entity_emb)
    enc = jnp.concatenate(
        [en.astype(ADT).reshape(N, NPASS, CHUNK).transpose(1, 0, 2).reshape(NPASS * N, CHUNK),
         jnp.zeros((ENC_ROWS - NPASS * N, CHUNK), ADT)])

    npad = EP - E
    srcp = jnp.concatenate([src, jnp.zeros((npad,), jnp.int32)]).reshape(EROWS, EBATCH)
    relp = jnp.concatenate([rel, jnp.zeros((npad,), jnp.int32)]).reshape(EROWS, EBATCH)
    dstp = jnp.concatenate([dst, jnp.full((npad,), SROWS, jnp.int32)]).reshape(EROWS, EBATCH)

    acc_all, hrows = _sc_phase1(enc, srcp, relp, dstp, h_idx.astype(jnp.int32), en)

    rblk = _reduce(acc_all, en)
    rall = rblk.transpose(0, 3, 1, 2).reshape(R, D, D)

    return _predict(hrows, r_idx.astype(jnp.int32).reshape(B, 1), rall)
